# baseline (speedup 1.0000x reference)
import functools
import math

import jax
import jax.numpy as jnp
from jax import lax
from jax.experimental import pallas as pl
from jax.experimental.pallas import tpu as pltpu
from jax.experimental.pallas import tpu_sc as plsc

_D = 64
_SEQ = 50
_BATCH = 4096
_NC = 2
_NS = 16
_NW = _NC * _NS
_N = _BATCH * _SEQ
_C = 128
_PER_W = _N // _NW
_NCHUNK = _PER_W // _C
_SCALE = math.sqrt(_D)
_L = 16


def _body(idx_hbm, par_hbm, tab_hbm, pe_hbm, out_hbm,
          idx_v, par_v, pe_v, buf0, buf1, obuf0, obuf1,
          gsem0, gsem1, ssem0, ssem1):
    cid = lax.axis_index("c")
    sid = lax.axis_index("s")
    wid = sid * _NC + cid
    base = wid * _PER_W
    pltpu.sync_copy(idx_hbm.at[pl.ds(base, _PER_W)], idx_v)
    pltpu.sync_copy(par_hbm.at[pl.ds(base, _PER_W)], par_v)
    pltpu.sync_copy(pe_hbm, pe_v)

    bufs = (buf0, buf1)
    obufs = (obuf0, obuf1)
    gsems = (gsem0, gsem1)
    ssems = (ssem0, ssem1)

    def start_gather(j, buf, gsem):
        pltpu.async_copy(tab_hbm.at[idx_v.at[pl.ds(j * _C, _C)]], buf, gsem)

    start_gather(0, buf0, gsem0)

    @pl.loop(0, _NCHUNK, step=2)
    def _pair(j0):
        for par in range(2):
            j = j0 + par
            buf, gsem, ssem, obuf = bufs[par], gsems[par], ssems[par], obufs[par]
            nbuf, ngsem = bufs[1 - par], gsems[1 - par]
            nssem, nobuf = ssems[1 - par], obufs[1 - par]

            @pl.when(j + 1 < _NCHUNK)
            def _start_next():
                @pl.when(j >= 1)
                def _drain_prev():
                    pltpu.make_async_copy(
                        nobuf,
                        out_hbm.at[pl.ds((base + (j - 1) * _C) * _D, _C * _D)],
                        nssem).wait()

                start_gather(j + 1, nbuf, ngsem)

            pltpu.make_async_copy(
                tab_hbm.at[idx_v.at[pl.ds(j * _C, _C)]], buf, gsem).wait()

            @pl.loop(0, _C, step=_L)
            def _rows(r0):
                roff = j * _C + r0
                lanes = lax.iota(jnp.int32, _L)
                rows = lanes + r0
                s16 = lax.rem(lanes + roff, _SEQ)
                pv = par_v[pl.ds(roff, _L)]
                cols0 = pv
                pebase = s16 * _D
                obase = rows * _D
                for t in range(_D):
                    v = plsc.load_gather(buf, [rows, cols0 + t])
                    pe16 = plsc.load_gather(pe_v, [pebase + t])
                    plsc.store_scatter(obuf, [obase + t], v * _SCALE + pe16)

            pltpu.async_copy(
                obuf, out_hbm.at[pl.ds((base + j * _C) * _D, _C * _D)], ssem)

    pltpu.make_async_copy(
        obuf0, out_hbm.at[pl.ds((base + (_NCHUNK - 2) * _C) * _D, _C * _D)],
        ssem0).wait()
    pltpu.make_async_copy(
        obuf1, out_hbm.at[pl.ds((base + (_NCHUNK - 1) * _C) * _D, _C * _D)],
        ssem1).wait()


@functools.partial(jax.jit, static_argnames=())
def kernel(enc_words, table, pe):
    enc = enc_words.reshape(_N).astype(jnp.int32)
    idx = enc >> 1
    par = (enc & 1) * _D
    tab2 = table.reshape(500000, 128)
    pe1 = pe.reshape(_SEQ * _D).astype(jnp.float32)
    mesh = plsc.VectorSubcoreMesh(core_axis_name="c", subcore_axis_name="s")
    out = pl.kernel(
        _body,
        out_type=jax.ShapeDtypeStruct((_N * _D,), jnp.float32),
        mesh=mesh,
        compiler_params=pltpu.CompilerParams(
            use_tc_tiling_on_sc=True, needs_layout_passes=False),
        scratch_types=[
            pltpu.VMEM((_PER_W,), jnp.int32),
            pltpu.VMEM((_PER_W,), jnp.int32),
            pltpu.VMEM((_SEQ * _D,), jnp.float32),
            pltpu.VMEM((_C, 128), jnp.float32),
            pltpu.VMEM((_C, 128), jnp.float32),
            pltpu.VMEM((_C * _D,), jnp.float32),
            pltpu.VMEM((_C * _D,), jnp.float32),
            pltpu.SemaphoreType.DMA,
            pltpu.SemaphoreType.DMA,
            pltpu.SemaphoreType.DMA,
            pltpu.SemaphoreType.DMA,
        ],
    )(idx, par, tab2, pe1)
    return out.reshape(_BATCH, _SEQ, _D)


# speedup vs baseline: 1.4467x; 1.4467x over previous
import functools
import math

import jax
import jax.numpy as jnp
from jax import lax
from jax.experimental import pallas as pl
from jax.experimental.pallas import tpu as pltpu
from jax.experimental.pallas import tpu_sc as plsc

_V = 1000000
_D = 64
_SEQ = 50
_BATCH = 4096
_NC = 2
_NS = 16
_NW = _NC * _NS
_N = _BATCH * _SEQ
_C = 128
_PER_W = _N // _NW
_NCHUNK = _PER_W // _C
_SCALE = math.sqrt(_D)
_L = 16

_PANEL = 1024
_TC_GRID = (_V + _PANEL - 1) // _PANEL


def _repack_body(t_ref, out_ref):
    out_ref[:, : _D] = t_ref[...].T


def _repack(tabT):
    return pl.pallas_call(
        _repack_body,
        grid=(_TC_GRID,),
        in_specs=[pl.BlockSpec((_D, _PANEL), lambda g: (0, g))],
        out_specs=pl.BlockSpec((_PANEL, 128), lambda g: (g, 0)),
        out_shape=jax.ShapeDtypeStruct((_V, 128), jnp.float32),
    )(tabT)


def _body(idx_hbm, tab_hbm, pe_hbm, out_hbm,
          idx_v, pe_v, buf0, buf1, obuf0, obuf1,
          gsem0, gsem1, ssem0, ssem1):
    cid = lax.axis_index("c")
    sid = lax.axis_index("s")
    wid = sid * _NC + cid
    base = wid * _PER_W
    pltpu.sync_copy(idx_hbm.at[pl.ds(base, _PER_W)], idx_v)
    pltpu.sync_copy(pe_hbm, pe_v)

    bufs = (buf0, buf1)
    obufs = (obuf0, obuf1)
    gsems = (gsem0, gsem1)
    ssems = (ssem0, ssem1)

    def start_gather(j, buf, gsem):
        pltpu.async_copy(tab_hbm.at[idx_v.at[pl.ds(j * _C, _C)]], buf, gsem)

    start_gather(0, buf0, gsem0)

    @pl.loop(0, _NCHUNK, step=2)
    def _pair(j0):
        for par in range(2):
            j = j0 + par
            buf, gsem, ssem, obuf = bufs[par], gsems[par], ssems[par], obufs[par]
            nbuf, ngsem = bufs[1 - par], gsems[1 - par]
            nssem, nobuf = ssems[1 - par], obufs[1 - par]

            @pl.when(j + 1 < _NCHUNK)
            def _start_next():
                @pl.when(j >= 1)
                def _drain_prev():
                    pltpu.make_async_copy(
                        nobuf,
                        out_hbm.at[pl.ds((base + (j - 1) * _C) * _D, _C * _D)],
                        nssem).wait()

                start_gather(j + 1, nbuf, ngsem)

            pltpu.make_async_copy(
                tab_hbm.at[idx_v.at[pl.ds(j * _C, _C)]], buf, gsem).wait()

            roff = j * _C

            @pl.loop(0, _C)
            def _row(r):
                s = lax.rem(roff + r, _SEQ)
                for t in range(_D // _L):
                    v = buf[r, pl.ds(t * _L, _L)]
                    pe16 = pe_v[pl.ds(s * _D + t * _L, _L)]
                    obuf[pl.ds(r * _D + t * _L, _L)] = v * _SCALE + pe16

            pltpu.async_copy(
                obuf, out_hbm.at[pl.ds((base + j * _C) * _D, _C * _D)], ssem)

    pltpu.make_async_copy(
        obuf0, out_hbm.at[pl.ds((base + (_NCHUNK - 2) * _C) * _D, _C * _D)],
        ssem0).wait()
    pltpu.make_async_copy(
        obuf1, out_hbm.at[pl.ds((base + (_NCHUNK - 1) * _C) * _D, _C * _D)],
        ssem1).wait()


@functools.partial(jax.jit, static_argnames=())
def kernel(enc_words, table, pe):
    idx = enc_words.reshape(_N).astype(jnp.int32)
    tab2 = _repack(table.T)
    pe1 = pe.reshape(_SEQ * _D).astype(jnp.float32)
    mesh = plsc.VectorSubcoreMesh(core_axis_name="c", subcore_axis_name="s")
    out = pl.kernel(
        _body,
        out_type=jax.ShapeDtypeStruct((_N * _D,), jnp.float32),
        mesh=mesh,
        compiler_params=pltpu.CompilerParams(
            use_tc_tiling_on_sc=True, needs_layout_passes=False),
        scratch_types=[
            pltpu.VMEM((_PER_W,), jnp.int32),
            pltpu.VMEM((_SEQ * _D,), jnp.float32),
            pltpu.VMEM((_C, 128), jnp.float32),
            pltpu.VMEM((_C, 128), jnp.float32),
            pltpu.VMEM((_C * _D,), jnp.float32),
            pltpu.VMEM((_C * _D,), jnp.float32),
            pltpu.SemaphoreType.DMA,
            pltpu.SemaphoreType.DMA,
            pltpu.SemaphoreType.DMA,
            pltpu.SemaphoreType.DMA,
        ],
    )(idx, tab2, pe1)
    return out.reshape(_BATCH, _SEQ, _D)


# speedup vs baseline: 1.8459x; 1.2760x over previous
import functools
import math

import jax
import jax.numpy as jnp
from jax import lax
from jax.experimental import pallas as pl
from jax.experimental.pallas import tpu as pltpu
from jax.experimental.pallas import tpu_sc as plsc

_V = 1000000
_D = 64
_SEQ = 50
_BATCH = 4096
_NC = 2
_NS = 16
_NW = _NC * _NS
_N = _BATCH * _SEQ
_C = 128
_PER_W = _N // _NW
_NCHUNK = _PER_W // _C
_SCALE = math.sqrt(_D)
_L = 16

_PANEL = 2048
_TC_GRID = (_V + _PANEL - 1) // _PANEL


def _repack_body(t_ref, out_ref):
    eye2 = jnp.concatenate(
        [jnp.eye(_D, dtype=jnp.float32)] * 2, axis=1)
    out_ref[...] = lax.dot_general(
        t_ref[...], eye2, (((0,), (0,)), ((), ())),
        preferred_element_type=jnp.float32)


def _repack(tabT):
    return pl.pallas_call(
        _repack_body,
        grid=(_TC_GRID,),
        in_specs=[pl.BlockSpec((_D, _PANEL), lambda g: (0, g))],
        out_specs=pl.BlockSpec((_PANEL, 128), lambda g: (g, 0)),
        out_shape=jax.ShapeDtypeStruct((_V, 128), jnp.float32),
    )(tabT)


def _body(idx_hbm, tab_hbm, pe_hbm, out_hbm,
          idx_v, pe_v, buf0, buf1, obuf0, obuf1,
          gsem0, gsem1, ssem0, ssem1):
    cid = lax.axis_index("c")
    sid = lax.axis_index("s")
    wid = sid * _NC + cid
    base = wid * _PER_W
    pltpu.sync_copy(idx_hbm.at[pl.ds(base, _PER_W)], idx_v)
    pltpu.sync_copy(pe_hbm, pe_v)

    bufs = (buf0, buf1)
    obufs = (obuf0, obuf1)
    gsems = (gsem0, gsem1)
    ssems = (ssem0, ssem1)

    def start_gather(j, buf, gsem):
        pltpu.async_copy(tab_hbm.at[idx_v.at[pl.ds(j * _C, _C)]], buf, gsem)

    start_gather(0, buf0, gsem0)

    @pl.loop(0, _NCHUNK, step=2)
    def _pair(j0):
        for par in range(2):
            j = j0 + par
            buf, gsem, ssem, obuf = bufs[par], gsems[par], ssems[par], obufs[par]
            nbuf, ngsem = bufs[1 - par], gsems[1 - par]
            nssem, nobuf = ssems[1 - par], obufs[1 - par]

            @pl.when(j + 1 < _NCHUNK)
            def _start_next():
                @pl.when(j >= 1)
                def _drain_prev():
                    pltpu.make_async_copy(
                        nobuf,
                        out_hbm.at[pl.ds((base + (j - 1) * _C) * _D, _C * _D)],
                        nssem).wait()

                start_gather(j + 1, nbuf, ngsem)

            pltpu.make_async_copy(
                tab_hbm.at[idx_v.at[pl.ds(j * _C, _C)]], buf, gsem).wait()

            roff = j * _C

            @pl.loop(0, _C)
            def _row(r):
                s = lax.rem(roff + r, _SEQ)
                for t in range(_D // _L):
                    v = buf[r, pl.ds(t * _L, _L)]
                    pe16 = pe_v[pl.ds(s * _D + t * _L, _L)]
                    obuf[pl.ds(r * _D + t * _L, _L)] = v * _SCALE + pe16

            pltpu.async_copy(
                obuf, out_hbm.at[pl.ds((base + j * _C) * _D, _C * _D)], ssem)

    pltpu.make_async_copy(
        obuf0, out_hbm.at[pl.ds((base + (_NCHUNK - 2) * _C) * _D, _C * _D)],
        ssem0).wait()
    pltpu.make_async_copy(
        obuf1, out_hbm.at[pl.ds((base + (_NCHUNK - 1) * _C) * _D, _C * _D)],
        ssem1).wait()


@functools.partial(jax.jit, static_argnames=())
def kernel(enc_words, table, pe):
    idx = enc_words.reshape(_N).astype(jnp.int32)
    tab2 = _repack(table.T)
    pe1 = pe.reshape(_SEQ * _D).astype(jnp.float32)
    mesh = plsc.VectorSubcoreMesh(core_axis_name="c", subcore_axis_name="s")
    out = pl.kernel(
        _body,
        out_type=jax.ShapeDtypeStruct((_N * _D,), jnp.float32),
        mesh=mesh,
        compiler_params=pltpu.CompilerParams(
            use_tc_tiling_on_sc=True, needs_layout_passes=False),
        scratch_types=[
            pltpu.VMEM((_PER_W,), jnp.int32),
            pltpu.VMEM((_SEQ * _D,), jnp.float32),
            pltpu.VMEM((_C, 128), jnp.float32),
            pltpu.VMEM((_C, 128), jnp.float32),
            pltpu.VMEM((_C * _D,), jnp.float32),
            pltpu.VMEM((_C * _D,), jnp.float32),
            pltpu.SemaphoreType.DMA,
            pltpu.SemaphoreType.DMA,
            pltpu.SemaphoreType.DMA,
            pltpu.SemaphoreType.DMA,
        ],
    )(idx, tab2, pe1)
    return out.reshape(_BATCH, _SEQ, _D)


# speedup vs baseline: 2.5344x; 1.3730x over previous
import functools
import math

import jax
import jax.numpy as jnp
from jax import lax
from jax.experimental import pallas as pl
from jax.experimental.pallas import tpu as pltpu
from jax.experimental.pallas import tpu_sc as plsc

_V = 1000000
_D = 64
_SEQ = 50
_BATCH = 4096
_NC = 2
_NS = 16
_NW = _NC * _NS
_N = _BATCH * _SEQ
_C = 128
_PER_W = _N // _NW
_NCHUNK = _PER_W // _C
_SCALE = math.sqrt(_D)
_L = 16

_PANEL = 8192
_TC_GRID = (_V + _PANEL - 1) // _PANEL


def _repack_body(t_ref, out_ref):
    eye2 = jnp.concatenate(
        [jnp.eye(_D, dtype=jnp.float32)] * 2, axis=1)
    out_ref[...] = lax.dot_general(
        t_ref[...], eye2, (((0,), (0,)), ((), ())),
        preferred_element_type=jnp.float32)


def _repack(tabT):
    return pl.pallas_call(
        _repack_body,
        grid=(_TC_GRID,),
        in_specs=[pl.BlockSpec((_D, _PANEL), lambda g: (0, g))],
        out_specs=pl.BlockSpec((_PANEL, 128), lambda g: (g, 0)),
        out_shape=jax.ShapeDtypeStruct((_V, 128), jnp.float32),
    )(tabT)


def _body(idx_hbm, tab_hbm, pe_hbm, out_hbm,
          idx_v, pe_v, buf0, buf1, obuf0, obuf1,
          gsem0, gsem1, ssem0, ssem1):
    cid = lax.axis_index("c")
    sid = lax.axis_index("s")
    wid = sid * _NC + cid
    base = wid * _PER_W
    pltpu.sync_copy(idx_hbm.at[pl.ds(base, _PER_W)], idx_v)
    pltpu.sync_copy(pe_hbm, pe_v)

    bufs = (buf0, buf1)
    obufs = (obuf0, obuf1)
    gsems = (gsem0, gsem1)
    ssems = (ssem0, ssem1)

    def start_gather(j, buf, gsem):
        pltpu.async_copy(tab_hbm.at[idx_v.at[pl.ds(j * _C, _C)]], buf, gsem)

    start_gather(0, buf0, gsem0)

    @pl.loop(0, _NCHUNK, step=2)
    def _pair(j0):
        for par in range(2):
            j = j0 + par
            buf, gsem, ssem, obuf = bufs[par], gsems[par], ssems[par], obufs[par]
            nbuf, ngsem = bufs[1 - par], gsems[1 - par]
            nssem, nobuf = ssems[1 - par], obufs[1 - par]

            @pl.when(j + 1 < _NCHUNK)
            def _start_next():
                @pl.when(j >= 1)
                def _drain_prev():
                    pltpu.make_async_copy(
                        nobuf,
                        out_hbm.at[pl.ds((base + (j - 1) * _C) * _D, _C * _D)],
                        nssem).wait()

                start_gather(j + 1, nbuf, ngsem)

            pltpu.make_async_copy(
                tab_hbm.at[idx_v.at[pl.ds(j * _C, _C)]], buf, gsem).wait()

            roff = j * _C

            @pl.loop(0, _C)
            def _row(r):
                s = lax.rem(roff + r, _SEQ)
                for t in range(_D // _L):
                    v = buf[r, pl.ds(t * _L, _L)]
                    pe16 = pe_v[pl.ds(s * _D + t * _L, _L)]
                    obuf[pl.ds(r * _D + t * _L, _L)] = v * _SCALE + pe16

            pltpu.async_copy(
                obuf, out_hbm.at[pl.ds((base + j * _C) * _D, _C * _D)], ssem)

    pltpu.make_async_copy(
        obuf0, out_hbm.at[pl.ds((base + (_NCHUNK - 2) * _C) * _D, _C * _D)],
        ssem0).wait()
    pltpu.make_async_copy(
        obuf1, out_hbm.at[pl.ds((base + (_NCHUNK - 1) * _C) * _D, _C * _D)],
        ssem1).wait()


@functools.partial(jax.jit, static_argnames=())
def kernel(enc_words, table, pe):
    idx = enc_words.reshape(_N).astype(jnp.int32)
    tab2 = _repack(table.T)
    pe1 = pe.reshape(_SEQ * _D).astype(jnp.float32)
    mesh = plsc.VectorSubcoreMesh(core_axis_name="c", subcore_axis_name="s")
    out = pl.kernel(
        _body,
        out_type=jax.ShapeDtypeStruct((_N * _D,), jnp.float32),
        mesh=mesh,
        compiler_params=pltpu.CompilerParams(
            use_tc_tiling_on_sc=True, needs_layout_passes=False),
        scratch_types=[
            pltpu.VMEM((_PER_W,), jnp.int32),
            pltpu.VMEM((_SEQ * _D,), jnp.float32),
            pltpu.VMEM((_C, 128), jnp.float32),
            pltpu.VMEM((_C, 128), jnp.float32),
            pltpu.VMEM((_C * _D,), jnp.float32),
            pltpu.VMEM((_C * _D,), jnp.float32),
            pltpu.SemaphoreType.DMA,
            pltpu.SemaphoreType.DMA,
            pltpu.SemaphoreType.DMA,
            pltpu.SemaphoreType.DMA,
        ],
    )(idx, tab2, pe1)
    return out.reshape(_BATCH, _SEQ, _D)


# speedup vs baseline: 2.6462x; 1.0441x over previous
import functools
import math

import jax
import jax.numpy as jnp
from jax import lax
from jax.experimental import pallas as pl
from jax.experimental.pallas import tpu as pltpu
from jax.experimental.pallas import tpu_sc as plsc

_V = 1000000
_D = 64
_SEQ = 50
_BATCH = 4096
_NC = 2
_NS = 16
_NW = _NC * _NS
_N = _BATCH * _SEQ
_C = 128
_PER_W = _N // _NW
_NCHUNK = _PER_W // _C
_SCALE = math.sqrt(_D)
_L = 16

_PANEL = 16384
_TC_GRID = (_V + _PANEL - 1) // _PANEL


def _repack_body(t_ref, out_ref):
    eye2 = jnp.concatenate(
        [jnp.eye(_D, dtype=jnp.float32)] * 2, axis=1)
    out_ref[...] = lax.dot_general(
        t_ref[...], eye2, (((0,), (0,)), ((), ())),
        preferred_element_type=jnp.float32)


def _repack(tabT):
    return pl.pallas_call(
        _repack_body,
        grid=(_TC_GRID,),
        in_specs=[pl.BlockSpec((_D, _PANEL), lambda g: (0, g))],
        out_specs=pl.BlockSpec((_PANEL, 128), lambda g: (g, 0)),
        out_shape=jax.ShapeDtypeStruct((_V, 128), jnp.float32),
    )(tabT)


def _body(idx_hbm, tab_hbm, pe_hbm, out_hbm,
          idx_v, pe_v, buf0, buf1, obuf0, obuf1,
          gsem0, gsem1, ssem0, ssem1):
    cid = lax.axis_index("c")
    sid = lax.axis_index("s")
    wid = sid * _NC + cid
    base = wid * _PER_W
    pltpu.sync_copy(idx_hbm.at[pl.ds(base, _PER_W)], idx_v)
    pltpu.sync_copy(pe_hbm, pe_v)

    bufs = (buf0, buf1)
    obufs = (obuf0, obuf1)
    gsems = (gsem0, gsem1)
    ssems = (ssem0, ssem1)

    def start_gather(j, buf, gsem):
        pltpu.async_copy(tab_hbm.at[idx_v.at[pl.ds(j * _C, _C)]], buf, gsem)

    start_gather(0, buf0, gsem0)

    @pl.loop(0, _NCHUNK, step=2)
    def _pair(j0):
        for par in range(2):
            j = j0 + par
            buf, gsem, ssem, obuf = bufs[par], gsems[par], ssems[par], obufs[par]
            nbuf, ngsem = bufs[1 - par], gsems[1 - par]
            nssem, nobuf = ssems[1 - par], obufs[1 - par]

            @pl.when(j + 1 < _NCHUNK)
            def _start_next():
                @pl.when(j >= 1)
                def _drain_prev():
                    pltpu.make_async_copy(
                        nobuf,
                        out_hbm.at[pl.ds((base + (j - 1) * _C) * _D, _C * _D)],
                        nssem).wait()

                start_gather(j + 1, nbuf, ngsem)

            pltpu.make_async_copy(
                tab_hbm.at[idx_v.at[pl.ds(j * _C, _C)]], buf, gsem).wait()

            roff = j * _C

            @pl.loop(0, _C)
            def _row(r):
                s = lax.rem(roff + r, _SEQ)
                for t in range(_D // _L):
                    v = buf[r, pl.ds(t * _L, _L)]
                    pe16 = pe_v[pl.ds(s * _D + t * _L, _L)]
                    obuf[pl.ds(r * _D + t * _L, _L)] = v * _SCALE + pe16

            pltpu.async_copy(
                obuf, out_hbm.at[pl.ds((base + j * _C) * _D, _C * _D)], ssem)

    pltpu.make_async_copy(
        obuf0, out_hbm.at[pl.ds((base + (_NCHUNK - 2) * _C) * _D, _C * _D)],
        ssem0).wait()
    pltpu.make_async_copy(
        obuf1, out_hbm.at[pl.ds((base + (_NCHUNK - 1) * _C) * _D, _C * _D)],
        ssem1).wait()


@functools.partial(jax.jit, static_argnames=())
def kernel(enc_words, table, pe):
    idx = enc_words.reshape(_N).astype(jnp.int32)
    tab2 = _repack(table.T)
    pe1 = pe.reshape(_SEQ * _D).astype(jnp.float32)
    mesh = plsc.VectorSubcoreMesh(core_axis_name="c", subcore_axis_name="s")
    out = pl.kernel(
        _body,
        out_type=jax.ShapeDtypeStruct((_N * _D,), jnp.float32),
        mesh=mesh,
        compiler_params=pltpu.CompilerParams(
            use_tc_tiling_on_sc=True, needs_layout_passes=False),
        scratch_types=[
            pltpu.VMEM((_PER_W,), jnp.int32),
            pltpu.VMEM((_SEQ * _D,), jnp.float32),
            pltpu.VMEM((_C, 128), jnp.float32),
            pltpu.VMEM((_C, 128), jnp.float32),
            pltpu.VMEM((_C * _D,), jnp.float32),
            pltpu.VMEM((_C * _D,), jnp.float32),
            pltpu.SemaphoreType.DMA,
            pltpu.SemaphoreType.DMA,
            pltpu.SemaphoreType.DMA,
            pltpu.SemaphoreType.DMA,
        ],
    )(idx, tab2, pe1)
    return out.reshape(_BATCH, _SEQ, _D)
